# baseline (speedup 1.0000x reference)
import functools

import jax
import jax.numpy as jnp
from jax import lax
from jax.experimental import pallas as pl
from jax.experimental.pallas import tpu as pltpu
from jax.experimental.pallas import tpu_sc as plsc

N = 10000
E = 320000
D = 128
F = 256

NC, NS, L = 2, 16, 16
NW = NC * NS
N_PAD = 10240
ROWS_W = N_PAD // NW
TRASH = ROWS_W
EPW = E // NW
GC = 80
NGC = EPW // GC
DCHUNK = 4000

_SC_MESH = plsc.VectorSubcoreMesh(
    core_axis_name="c", subcore_axis_name="s", num_cores=NC, num_subcores=NS)


def _worker_id():
    return lax.axis_index("s") * NC + lax.axis_index("c")


def _prep_body(x_ref, pos_ref, wh1_ref, bh1_ref, wh2_ref, bh2_ref,
               wx_ref, wr_ref, bf1_ref, u_ref, v_ref):
    x = x_ref[...]
    pos = pos_ref[...]
    h = jnp.maximum(
        jnp.dot(x, wh1_ref[...], preferred_element_type=jnp.float32)
        + bh1_ref[...], 0.0)
    delta = jnp.dot(h, wh2_ref[...], preferred_element_type=jnp.float32) \
        + bh2_ref[...]
    wr = wr_ref[...]
    u_ref[...] = jnp.dot(x, wx_ref[...], preferred_element_type=jnp.float32) \
        + jnp.dot(pos, wr, preferred_element_type=jnp.float32)
    v_ref[...] = jnp.dot(delta - pos, wr,
                         preferred_element_type=jnp.float32) + bf1_ref[...]


def _prep(x_p, pos_p, wh1, bh1, wh2p, bh2p, wx, wrp, bf1):
    rb = 1024
    grid = (N_PAD // rb,)
    row_spec = lambda w: pl.BlockSpec((rb, w), lambda i: (i, 0))
    full = lambda a: pl.BlockSpec(a.shape, lambda i: (0,) * a.ndim)
    return pl.pallas_call(
        _prep_body,
        grid=grid,
        in_specs=[row_spec(D), row_spec(D), full(wh1), full(bh1), full(wh2p),
                  full(bh2p), full(wx), full(wrp), full(bf1)],
        out_specs=[row_spec(F), row_spec(F)],
        out_shape=[jax.ShapeDtypeStruct((N_PAD, F), jnp.float32),
                   jax.ShapeDtypeStruct((N_PAD, F), jnp.float32)],
    )(x_p, pos_p, wh1, bh1, wh2p, bh2p, wx, wrp, bf1)


@functools.partial(
    pl.kernel,
    out_type=(jax.ShapeDtypeStruct((E, F), jnp.float32),
              jax.ShapeDtypeStruct((E, F), jnp.float32)),
    mesh=_SC_MESH,
    scratch_types=[
        pltpu.VMEM((GC,), jnp.int32),
        pltpu.VMEM((GC,), jnp.int32),
        pltpu.VMEM((GC, F), jnp.float32),
        pltpu.VMEM((GC, F), jnp.float32),
        pltpu.SemaphoreType.DMA,
        pltpu.SemaphoreType.DMA,
    ],
    compiler_params=pltpu.CompilerParams(needs_layout_passes=False),
)
def _gather(src_hbm, dst_hbm, u_hbm, v_hbm, s_out, t_out,
            sidx, didx, srows, trows, sem1, sem2):
    base0 = _worker_id() * EPW

    def step(g, carry):
        base = base0 + g * GC
        pltpu.sync_copy(src_hbm.at[pl.ds(base, GC)], sidx)
        pltpu.sync_copy(dst_hbm.at[pl.ds(base, GC)], didx)
        c1 = pltpu.async_copy(u_hbm.at[sidx], srows, sem1)
        c2 = pltpu.async_copy(v_hbm.at[didx], trows, sem2)
        c1.wait()
        c2.wait()
        pltpu.sync_copy(srows, s_out.at[pl.ds(base, GC)])
        pltpu.sync_copy(trows, t_out.at[pl.ds(base, GC)])
        return carry

    lax.fori_loop(0, NGC, step, 0)


def _edge_body(s_ref, t_ref, w2_ref, b2_ref, e_ref):
    h = jnp.maximum(s_ref[...] + t_ref[...], 0.0)
    e_ref[...] = jnp.dot(h, w2_ref[...],
                         preferred_element_type=jnp.float32) + b2_ref[...]


def _edge_mlp(s, t, w2, b2):
    eb = 512
    grid = (E // eb,)
    row_spec = pl.BlockSpec((eb, F), lambda i: (i, 0))
    full = lambda a: pl.BlockSpec(a.shape, lambda i: (0,) * a.ndim)
    return pl.pallas_call(
        _edge_body,
        grid=grid,
        in_specs=[row_spec, row_spec, full(w2), full(b2)],
        out_specs=row_spec,
        out_shape=jax.ShapeDtypeStruct((E, F), jnp.float32),
    )(s, t, w2, b2)


DCH = 6400
NBUF = 4


@functools.partial(
    pl.kernel,
    out_type=jax.ShapeDtypeStruct((N_PAD, F), jnp.float32),
    mesh=_SC_MESH,
    scratch_types=[
        pltpu.VMEM((DCH,), jnp.int32),
        pltpu.VMEM((DCH + L,), jnp.int32),
        pltpu.VMEM((DCH + L,), jnp.int32),
        [pltpu.VMEM((L, F), jnp.float32)] * NBUF,
        pltpu.VMEM((ROWS_W + 1, F), jnp.float32),
        [pltpu.SemaphoreType.DMA] * NBUF,
    ],
    compiler_params=pltpu.CompilerParams(needs_layout_passes=False),
)
def _segmax(dst_hbm, e_hbm, agg_out, dbuf, ids, dloc, rows, agg_l, sems):
    wid = _worker_id()
    lo = wid * ROWS_W
    hi = lo + ROWS_W
    neg_inf = jnp.full((L,), -jnp.inf, jnp.float32)

    def init_row(i, carry):
        for t in range(F // L):
            agg_l[i, pl.ds(t * L, L)] = neg_inf
        return carry

    lax.fori_loop(0, ROWS_W + 1, init_row, 0)

    def issue(g, b):
        pltpu.async_copy(e_hbm.at[ids.at[pl.ds(g * L, L)]], rows[b], sems[b])

    def drain(g, b):
        pltpu.make_async_copy(e_hbm.at[ids.at[pl.ds(g * L, L)]], rows[b],
                              sems[b]).wait()

    def apply(g, b):
        dvec = dloc[pl.ds(g * L, L)]
        for j in range(L):
            dj = dvec[j]
            for t in range(F // L):
                sl = pl.ds(t * L, L)
                agg_l[dj, sl] = jnp.maximum(agg_l[dj, sl], rows[b][j, sl])

    def chunk(cidx, carry):
        cbase = cidx * DCH
        pltpu.sync_copy(dst_hbm.at[pl.ds(cbase, DCH)], dbuf)

        def scan16(i, off):
            d = dbuf[pl.ds(i * L, L)]
            m = (d >= lo) & (d < hi)
            cnt = plsc.all_reduce_population_count(m)[0]

            @pl.when(cnt > 0)
            def _():
                eid = cbase + i * L + lax.iota(jnp.int32, L)
                sk, sv, _om = plsc.sort_key_val(d - lo, eid, mask=m)
                posv = off + lax.iota(jnp.int32, L)
                plsc.store_scatter(ids, [posv], sv)
                plsc.store_scatter(dloc, [posv], sk)

            return off + cnt

        off = lax.fori_loop(0, DCH // L, scan16, jnp.int32(0))
        padp = off + lax.iota(jnp.int32, L)
        plsc.store_scatter(ids, [padp], jnp.zeros((L,), jnp.int32))
        plsc.store_scatter(dloc, [padp], jnp.full((L,), TRASH, jnp.int32))
        ngrp = (off + L - 1) // L

        for b in range(NBUF):
            @pl.when(b < ngrp)
            def _():
                issue(b, b)

        def quad(p, carry2):
            for b in range(NBUF):
                g = NBUF * p + b

                @pl.when(g < ngrp)
                def _():
                    drain(g, b)
                    apply(g, b)

                    @pl.when(g + NBUF < ngrp)
                    def _():
                        issue(g + NBUF, b)
            return carry2

        lax.fori_loop(0, (ngrp + NBUF - 1) // NBUF, quad, 0)
        return carry

    lax.fori_loop(0, E // DCH, chunk, 0)
    pltpu.sync_copy(agg_l.at[pl.ds(0, ROWS_W)], agg_out.at[pl.ds(lo, ROWS_W)])


def _final_body(x_ref, agg_ref, wg1_ref, bg1_ref, wg2_ref, bg2_ref, o_ref):
    a = agg_ref[...]
    a = jnp.where(jnp.isfinite(a), a, 0.0)
    g = jnp.maximum(
        jnp.dot(a, wg1_ref[...], preferred_element_type=jnp.float32)
        + bg1_ref[...], 0.0)
    o_ref[...] = x_ref[...] + jnp.dot(
        g, wg2_ref[...], preferred_element_type=jnp.float32) + bg2_ref[...]


def _final(x, agg, wg1, bg1, wg2, bg2):
    rb = 1000
    grid = (N // rb,)
    full = lambda a: pl.BlockSpec(a.shape, lambda i: (0,) * a.ndim)
    return pl.pallas_call(
        _final_body,
        grid=grid,
        in_specs=[pl.BlockSpec((rb, D), lambda i: (i, 0)),
                  pl.BlockSpec((rb, F), lambda i: (i, 0)),
                  full(wg1), full(bg1), full(wg2), full(bg2)],
        out_specs=pl.BlockSpec((rb, D), lambda i: (i, 0)),
        out_shape=jax.ShapeDtypeStruct((N, D), jnp.float32),
    )(x, agg, wg1, bg1, wg2, bg2)


def kernel(x, pos, edge_index, W_h1, b_h1, W_h2, b_h2,
           W_f1, b_f1, W_f2, b_f2, W_g1, b_g1, W_g2, b_g2):
    src = edge_index[0]
    dst = edge_index[1]

    x_p = jnp.pad(x, ((0, N_PAD - N), (0, 0)))
    pos_p = jnp.pad(pos, ((0, N_PAD - N), (0, D - 3)))
    wh2p = jnp.pad(W_h2, ((0, 0), (0, D - 3)))
    bh2p = jnp.pad(b_h2, (0, D - 3)).reshape(1, D)
    wrp = jnp.pad(W_f1[:3], ((0, D - 3), (0, 0)))
    wx = W_f1[3:]

    u, v = _prep(x_p, pos_p, W_h1, b_h1.reshape(1, -1), wh2p, bh2p,
                 wx, wrp, b_f1.reshape(1, F))
    s, t = _gather(src, dst, u, v)
    e = _edge_mlp(s, t, W_f2, b_f2.reshape(1, F))
    agg = _segmax(dst, e)
    return _final(x, agg[:N], W_g1, b_g1.reshape(1, F), W_g2,
                  b_g2.reshape(1, D))


# speedup vs baseline: 1.0694x; 1.0694x over previous
import functools

import jax
import jax.numpy as jnp
from jax import lax
from jax.experimental import pallas as pl
from jax.experimental.pallas import tpu as pltpu
from jax.experimental.pallas import tpu_sc as plsc

N = 10000
E = 320000
D = 128
F = 256

NC, NS, L = 2, 16, 16
NW = NC * NS
N_PAD = 10240
ROWS_W = N_PAD // NW
TRASH = ROWS_W
EPW = E // NW
GC = 80
NGC = EPW // GC
DCHUNK = 4000

_SC_MESH = plsc.VectorSubcoreMesh(
    core_axis_name="c", subcore_axis_name="s", num_cores=NC, num_subcores=NS)


def _worker_id():
    return lax.axis_index("s") * NC + lax.axis_index("c")


def _prep_body(x_ref, pos_ref, wh1_ref, bh1_ref, wh2_ref, bh2_ref,
               wx_ref, wr_ref, bf1_ref, u_ref, v_ref):
    x = x_ref[...]
    pos = pos_ref[...]
    h = jnp.maximum(
        jnp.dot(x, wh1_ref[...], preferred_element_type=jnp.float32)
        + bh1_ref[...], 0.0)
    delta = jnp.dot(h, wh2_ref[...], preferred_element_type=jnp.float32) \
        + bh2_ref[...]
    wr = wr_ref[...]
    u_ref[...] = jnp.dot(x, wx_ref[...], preferred_element_type=jnp.float32) \
        + jnp.dot(pos, wr, preferred_element_type=jnp.float32)
    v_ref[...] = jnp.dot(delta - pos, wr,
                         preferred_element_type=jnp.float32) + bf1_ref[...]


def _prep(x_p, pos_p, wh1, bh1, wh2p, bh2p, wx, wrp, bf1):
    rb = 1024
    grid = (N_PAD // rb,)
    row_spec = lambda w: pl.BlockSpec((rb, w), lambda i: (i, 0))
    full = lambda a: pl.BlockSpec(a.shape, lambda i: (0,) * a.ndim)
    return pl.pallas_call(
        _prep_body,
        grid=grid,
        in_specs=[row_spec(D), row_spec(D), full(wh1), full(bh1), full(wh2p),
                  full(bh2p), full(wx), full(wrp), full(bf1)],
        out_specs=[row_spec(F), row_spec(F)],
        out_shape=[jax.ShapeDtypeStruct((N_PAD, F), jnp.float32),
                   jax.ShapeDtypeStruct((N_PAD, F), jnp.float32)],
    )(x_p, pos_p, wh1, bh1, wh2p, bh2p, wx, wrp, bf1)


@functools.partial(
    pl.kernel,
    out_type=(jax.ShapeDtypeStruct((E, F), jnp.float32),
              jax.ShapeDtypeStruct((E, F), jnp.float32)),
    mesh=_SC_MESH,
    scratch_types=[
        pltpu.VMEM((GC,), jnp.int32),
        pltpu.VMEM((GC,), jnp.int32),
        pltpu.VMEM((GC, F), jnp.float32),
        pltpu.VMEM((GC, F), jnp.float32),
        pltpu.SemaphoreType.DMA,
        pltpu.SemaphoreType.DMA,
    ],
    compiler_params=pltpu.CompilerParams(needs_layout_passes=False),
)
def _gather(src_hbm, dst_hbm, u_hbm, v_hbm, s_out, t_out,
            sidx, didx, srows, trows, sem1, sem2):
    base0 = _worker_id() * EPW

    def step(g, carry):
        base = base0 + g * GC
        pltpu.sync_copy(src_hbm.at[pl.ds(base, GC)], sidx)
        pltpu.sync_copy(dst_hbm.at[pl.ds(base, GC)], didx)
        c1 = pltpu.async_copy(u_hbm.at[sidx], srows, sem1)
        c2 = pltpu.async_copy(v_hbm.at[didx], trows, sem2)
        c1.wait()
        c2.wait()
        pltpu.sync_copy(srows, s_out.at[pl.ds(base, GC)])
        pltpu.sync_copy(trows, t_out.at[pl.ds(base, GC)])
        return carry

    lax.fori_loop(0, NGC, step, 0)


def _edge_body(s_ref, t_ref, w2_ref, b2_ref, e_ref):
    h = jnp.maximum(s_ref[...] + t_ref[...], 0.0)
    e_ref[...] = jnp.dot(h, w2_ref[...],
                         preferred_element_type=jnp.float32) + b2_ref[...]


def _edge_mlp(s, t, w2, b2):
    eb = 512
    grid = (E // eb,)
    row_spec = pl.BlockSpec((eb, F), lambda i: (i, 0))
    full = lambda a: pl.BlockSpec(a.shape, lambda i: (0,) * a.ndim)
    return pl.pallas_call(
        _edge_body,
        grid=grid,
        in_specs=[row_spec, row_spec, full(w2), full(b2)],
        out_specs=row_spec,
        out_shape=jax.ShapeDtypeStruct((E, F), jnp.float32),
    )(s, t, w2, b2)


DCH = 6400
NBUF = 4


@functools.partial(
    pl.kernel,
    out_type=jax.ShapeDtypeStruct((N_PAD, F), jnp.float32),
    mesh=_SC_MESH,
    scratch_types=[
        pltpu.VMEM((DCH,), jnp.int32),
        pltpu.VMEM((DCH + L,), jnp.int32),
        pltpu.VMEM((DCH + L,), jnp.int32),
        [pltpu.VMEM((L, F), jnp.float32)] * NBUF,
        pltpu.VMEM((ROWS_W + 1, F), jnp.float32),
        [pltpu.SemaphoreType.DMA] * NBUF,
    ],
    compiler_params=pltpu.CompilerParams(needs_layout_passes=False),
)
def _segmax(dst_hbm, e_hbm, agg_out, dbuf, ids, dloc, rows, agg_l, sems):
    wid = _worker_id()
    lo = wid * ROWS_W
    hi = lo + ROWS_W
    neg_inf = jnp.full((L,), -jnp.inf, jnp.float32)

    def init_row(i, carry):
        for t in range(F // L):
            agg_l[i, pl.ds(t * L, L)] = neg_inf
        return carry

    lax.fori_loop(0, ROWS_W + 1, init_row, 0)

    def issue(g, b):
        pltpu.async_copy(e_hbm.at[ids.at[pl.ds(g * L, L)]], rows[b], sems[b])

    def drain(g, b):
        pltpu.make_async_copy(e_hbm.at[ids.at[pl.ds(g * L, L)]], rows[b],
                              sems[b]).wait()

    def apply(g, b):
        dvec = dloc[pl.ds(g * L, L)]
        for j in range(L):
            dj = dvec[j]
            for t in range(F // L):
                sl = pl.ds(t * L, L)
                agg_l[dj, sl] = jnp.maximum(agg_l[dj, sl], rows[b][j, sl])

    def chunk(cidx, carry):
        cbase = cidx * DCH
        pltpu.sync_copy(dst_hbm.at[pl.ds(cbase, DCH)], dbuf)

        def scan16(i, off):
            d = dbuf[pl.ds(i * L, L)]
            m = (d >= lo) & (d < hi)
            eid = cbase + i * L + lax.iota(jnp.int32, L)
            sk, sv, _om = plsc.sort_key_val(d - lo, eid, mask=m)
            posv = off + lax.iota(jnp.int32, L)
            plsc.store_scatter(ids, [posv], sv)
            plsc.store_scatter(dloc, [posv], sk)
            cnt = plsc.all_reduce_population_count(m)
            return off + cnt[0]

        off = lax.fori_loop(0, DCH // L, scan16, jnp.int32(0))
        padp = off + lax.iota(jnp.int32, L)
        plsc.store_scatter(ids, [padp], jnp.zeros((L,), jnp.int32))
        plsc.store_scatter(dloc, [padp], jnp.full((L,), TRASH, jnp.int32))
        ngrp = (off + L - 1) // L

        for b in range(NBUF):
            @pl.when(b < ngrp)
            def _():
                issue(b, b)

        def quad(p, carry2):
            for b in range(NBUF):
                g = NBUF * p + b

                @pl.when(g < ngrp)
                def _():
                    drain(g, b)
                    apply(g, b)

                    @pl.when(g + NBUF < ngrp)
                    def _():
                        issue(g + NBUF, b)
            return carry2

        lax.fori_loop(0, (ngrp + NBUF - 1) // NBUF, quad, 0)
        return carry

    lax.fori_loop(0, E // DCH, chunk, 0)
    pltpu.sync_copy(agg_l.at[pl.ds(0, ROWS_W)], agg_out.at[pl.ds(lo, ROWS_W)])


def _final_body(x_ref, agg_ref, wg1_ref, bg1_ref, wg2_ref, bg2_ref, o_ref):
    a = agg_ref[...]
    a = jnp.where(jnp.isfinite(a), a, 0.0)
    g = jnp.maximum(
        jnp.dot(a, wg1_ref[...], preferred_element_type=jnp.float32)
        + bg1_ref[...], 0.0)
    o_ref[...] = x_ref[...] + jnp.dot(
        g, wg2_ref[...], preferred_element_type=jnp.float32) + bg2_ref[...]


def _final(x, agg, wg1, bg1, wg2, bg2):
    rb = 1000
    grid = (N // rb,)
    full = lambda a: pl.BlockSpec(a.shape, lambda i: (0,) * a.ndim)
    return pl.pallas_call(
        _final_body,
        grid=grid,
        in_specs=[pl.BlockSpec((rb, D), lambda i: (i, 0)),
                  pl.BlockSpec((rb, F), lambda i: (i, 0)),
                  full(wg1), full(bg1), full(wg2), full(bg2)],
        out_specs=pl.BlockSpec((rb, D), lambda i: (i, 0)),
        out_shape=jax.ShapeDtypeStruct((N, D), jnp.float32),
    )(x, agg, wg1, bg1, wg2, bg2)


def kernel(x, pos, edge_index, W_h1, b_h1, W_h2, b_h2,
           W_f1, b_f1, W_f2, b_f2, W_g1, b_g1, W_g2, b_g2):
    src = edge_index[0]
    dst = edge_index[1]

    x_p = jnp.pad(x, ((0, N_PAD - N), (0, 0)))
    pos_p = jnp.pad(pos, ((0, N_PAD - N), (0, D - 3)))
    wh2p = jnp.pad(W_h2, ((0, 0), (0, D - 3)))
    bh2p = jnp.pad(b_h2, (0, D - 3)).reshape(1, D)
    wrp = jnp.pad(W_f1[:3], ((0, D - 3), (0, 0)))
    wx = W_f1[3:]

    u, v = _prep(x_p, pos_p, W_h1, b_h1.reshape(1, -1), wh2p, bh2p,
                 wx, wrp, b_f1.reshape(1, F))
    s, t = _gather(src, dst, u, v)
    e = _edge_mlp(s, t, W_f2, b_f2.reshape(1, F))
    agg = _segmax(dst, e)
    return _final(x, agg[:N], W_g1, b_g1.reshape(1, F), W_g2,
                  b_g2.reshape(1, D))
